# Initial kernel scaffold; baseline (speedup 1.0000x reference)
import jax
import jax.numpy as jnp
from jax.experimental import pallas as pl


def kernel(x, edge_index, Wp, bp, cw0, cb0, cw1, cb1, cw2, cb2, cw3, cb3, pw0, pb0, pw1, pb1, pw2, pb2, rw0, rb0, rw1, rb1):
    raise NotImplementedError("write your pallas kernel here")


# speedup vs baseline: 2.2487x; 2.2487x over previous
import functools

import jax
import jax.numpy as jnp
from jax.experimental import pallas as pl
from jax.experimental.pallas import tpu as pltpu

N = 50000
E = 800000
D_IN = 41
H = 192

ROWS = 512
NP = 50176

_INTERPRET = False


def _row_spec(width):
    return pl.BlockSpec((ROWS, width), lambda i: (i, 0))


def _full_spec(shape):
    nd = len(shape)
    return pl.BlockSpec(shape, lambda i: (0,) * nd)


def _tc_call(body, in_specs, out_width_list):
    return pl.pallas_call(
        body,
        grid=(NP // ROWS,),
        in_specs=in_specs,
        out_specs=[_row_spec(w) for w in out_width_list],
        out_shape=[jax.ShapeDtypeStruct((NP, w), jnp.float32) for w in out_width_list],
        interpret=_INTERPRET,
    )


def _in_body(x_ref, wp_ref, bp_ref, cw_ref, dinv_ref, h_ref, g_ref):
    h = jnp.dot(x_ref[...], wp_ref[...], preferred_element_type=jnp.float32) + bp_ref[...]
    h_ref[...] = h
    g_ref[...] = jnp.dot(h, cw_ref[...], preferred_element_type=jnp.float32) * dinv_ref[...]


def _mid_body(acc_ref, hprev_ref, dinv_ref, cb_ref, cw_ref, h_ref, g_ref):
    conv = acc_ref[...] * dinv_ref[...] + cb_ref[...]
    h = hprev_ref[...] + jnp.maximum(conv, 0.0)
    h_ref[...] = h
    g_ref[...] = jnp.dot(h, cw_ref[...], preferred_element_type=jnp.float32) * dinv_ref[...]


def _out_body(acc_ref, hprev_ref, dinv_ref, cb_ref,
              pw0_ref, pb0_ref, pw1_ref, pb1_ref, pw2_ref, pb2_ref,
              rw0_ref, rb0_ref, rw1_ref, rb1_ref, out_ref):
    conv = acc_ref[...] * dinv_ref[...] + cb_ref[...]
    h = hprev_ref[...] + jnp.maximum(conv, 0.0)
    p = jnp.maximum(jnp.dot(h, pw0_ref[...], preferred_element_type=jnp.float32) + pb0_ref[...], 0.0)
    p = jnp.maximum(jnp.dot(p, pw1_ref[...], preferred_element_type=jnp.float32) + pb1_ref[...], 0.0)
    p = jnp.dot(p, pw2_ref[...], preferred_element_type=jnp.float32) + pb2_ref[...]
    r = jnp.maximum(jnp.dot(h, rw0_ref[...], preferred_element_type=jnp.float32) + rb0_ref[...], 0.0)
    r = jax.nn.sigmoid(jnp.dot(r, rw1_ref[...], preferred_element_type=jnp.float32) + rb1_ref[...])
    nrm = jnp.sqrt(jnp.sum(p * p, axis=-1, keepdims=True))
    out_ref[...] = p / (nrm + 1e-8) * r


def _segment_sum_placeholder(g, src, dst):
    msg = jnp.take(g, src, axis=0)
    return g + jax.ops.segment_sum(msg, dst, num_segments=NP)


def kernel(x, edge_index, Wp, bp, cw0, cb0, cw1, cb1, cw2, cb2, cw3, cb3,
           pw0, pb0, pw1, pb1, pw2, pb2, rw0, rb0, rw1, rb1):
    src = edge_index[0]
    dst = edge_index[1]

    deg = 1.0 + jax.ops.segment_sum(jnp.ones((E,), jnp.float32), dst, num_segments=NP)
    dinv = jax.lax.rsqrt(deg)
    dinv = dinv.at[N:].set(0.0)[:, None]

    xp = jnp.pad(x, ((0, NP - N), (0, 128 - D_IN)))
    wpp = jnp.pad(Wp, ((0, 128 - D_IN), (0, 0)))

    h, g = _tc_call(
        _in_body,
        [_row_spec(128), _full_spec((128, H)), _full_spec((H,)),
         _full_spec((H, H)), _row_spec(1)],
        [H, H],
    )(xp, wpp, bp, cw0, dinv)

    for cw_next, cb in ((cw1, cb0), (cw2, cb1), (cw3, cb2)):
        acc = _segment_sum_placeholder(g, src, dst)
        h, g = _tc_call(
            _mid_body,
            [_row_spec(H), _row_spec(H), _row_spec(1), _full_spec((H,)),
             _full_spec((H, H))],
            [H, H],
        )(acc, h, dinv, cb, cw_next)
    acc = _segment_sum_placeholder(g, src, dst)

    (coords,) = _tc_call(
        _out_body,
        [_row_spec(H), _row_spec(H), _row_spec(1), _full_spec((H,)),
         _full_spec((H, H)), _full_spec((H,)),
         _full_spec((H, H // 2)), _full_spec((H // 2,)),
         _full_spec((H // 2, 2)), _full_spec((2,)),
         _full_spec((H, H // 2)), _full_spec((H // 2,)),
         _full_spec((H // 2, 1)), _full_spec((1,))],
        [2],
    )(acc, h, dinv, cb3, pw0, pb0, pw1, pb1, pw2, pb2, rw0, rb0, rw1, rb1)
    return coords[:N]


# speedup vs baseline: 7.9656x; 3.5422x over previous
import functools

import jax
import jax.numpy as jnp
from jax.experimental import pallas as pl
from jax.experimental.pallas import tpu as pltpu
from jax.experimental.pallas import tpu_sc as plsc

N = 50000
E = 800000
D_IN = 41
H = 192

ROWS = 512
NP = 50688

NSC = 2
NTILE = 16
NPASS = 3
NCHUNK = NSC * NPASS
C = 8448
RPT = C // NTILE
B = 128
EP = E + 2 * B

_INTERPRET = False


def _row_spec(width):
    return pl.BlockSpec((ROWS, width), lambda i: (i, 0))


def _full_spec(shape):
    nd = len(shape)
    return pl.BlockSpec(shape, lambda i: (0,) * nd)


def _tc_call(body, in_specs, out_width_list):
    return pl.pallas_call(
        body,
        grid=(NP // ROWS,),
        in_specs=in_specs,
        out_specs=[_row_spec(w) for w in out_width_list],
        out_shape=[jax.ShapeDtypeStruct((NP, w), jnp.float32) for w in out_width_list],
        interpret=_INTERPRET,
    )


def _in_body(x_ref, wp_ref, bp_ref, cw_ref, dinv_ref, h_ref, g_ref):
    h = jnp.dot(x_ref[...], wp_ref[...], preferred_element_type=jnp.float32) + bp_ref[...]
    h_ref[...] = h
    g_ref[...] = jnp.dot(h, cw_ref[...], preferred_element_type=jnp.float32) * dinv_ref[...]


def _mid_body(acc_ref, hprev_ref, dinv_ref, cb_ref, cw_ref, h_ref, g_ref):
    conv = acc_ref[...] * dinv_ref[...] + cb_ref[...]
    h = hprev_ref[...] + jnp.maximum(conv, 0.0)
    h_ref[...] = h
    g_ref[...] = jnp.dot(h, cw_ref[...], preferred_element_type=jnp.float32) * dinv_ref[...]


def _out_body(acc_ref, hprev_ref, dinv_ref, cb_ref,
              pw0_ref, pb0_ref, pw1_ref, pb1_ref, pw2_ref, pb2_ref,
              rw0_ref, rb0_ref, rw1_ref, rb1_ref, out_ref):
    conv = acc_ref[...] * dinv_ref[...] + cb_ref[...]
    h = hprev_ref[...] + jnp.maximum(conv, 0.0)
    p = jnp.maximum(jnp.dot(h, pw0_ref[...], preferred_element_type=jnp.float32) + pb0_ref[...], 0.0)
    p = jnp.maximum(jnp.dot(p, pw1_ref[...], preferred_element_type=jnp.float32) + pb1_ref[...], 0.0)
    p = jnp.dot(p, pw2_ref[...], preferred_element_type=jnp.float32) + pb2_ref[...]
    r = jnp.maximum(jnp.dot(h, rw0_ref[...], preferred_element_type=jnp.float32) + rb0_ref[...], 0.0)
    r = jax.nn.sigmoid(jnp.dot(r, rw1_ref[...], preferred_element_type=jnp.float32) + rb1_ref[...])
    nrm = jnp.sqrt(jnp.sum(p * p, axis=-1, keepdims=True))
    out_ref[...] = p / (nrm + 1e-8) * r


def _sc_body(g_hbm, srcp_hbm, dstp_hbm, meta_hbm, out_hbm,
             acc, meta_v, sidx, didx, lidx, rows, sem):
    cid = jax.lax.axis_index("c")
    sid = jax.lax.axis_index("s")
    r0 = sid * RPT
    for p in range(NPASS):
        chunk = p * NSC + cid
        base = chunk * C
        pltpu.sync_copy(g_hbm.at[pl.ds(base + r0, RPT)], acc.at[pl.ds(r0, RPT)])
        pltpu.sync_copy(meta_hbm.at[chunk * NTILE + sid], meta_v)
        plsc.subcore_barrier()
        mv = meta_v[...]
        astart = mv[0]
        estart = mv[1]
        eend = mv[2]
        nb = mv[3]

        def body(b, carry):
            off = pl.multiple_of(astart + b * B, 8)
            pltpu.sync_copy(srcp_hbm.at[pl.ds(off, B)], sidx)
            pltpu.sync_copy(dstp_hbm.at[pl.ds(off, B)], didx)
            pltpu.async_copy(g_hbm.at[sidx], rows, sem).wait()
            for j in range(B // 16):
                dv = didx[pl.ds(j * 16, 16)]
                pos = off + j * 16 + jax.lax.iota(jnp.int32, 16)
                valid = (pos >= estart) & (pos < eend)
                lidx[pl.ds(j * 16, 16)] = jnp.where(valid, dv - base, C)
            pltpu.sync_copy(rows, acc.at[lidx], add=True)
            return carry

        jax.lax.fori_loop(0, nb, body, 0)
        plsc.subcore_barrier()
        pltpu.sync_copy(acc.at[pl.ds(r0, RPT)], out_hbm.at[pl.ds(base + r0, RPT)])


_sc_segsum = functools.partial(
    pl.kernel,
    out_type=jax.ShapeDtypeStruct((NP, H), jnp.float32),
    mesh=plsc.VectorSubcoreMesh(core_axis_name="c", subcore_axis_name="s"),
    scratch_types=[
        pltpu.VMEM_SHARED((C + 8, H), jnp.float32),
        pltpu.VMEM((16,), jnp.int32),
        pltpu.VMEM((B,), jnp.int32),
        pltpu.VMEM((B,), jnp.int32),
        pltpu.VMEM((B,), jnp.int32),
        pltpu.VMEM((B, H), jnp.float32),
        pltpu.SemaphoreType.DMA,
    ],
    compiler_params=pltpu.CompilerParams(use_tc_tiling_on_sc=False),
)(_sc_body)


def _partition_edges(src, dst):
    dst_s, src_s = jax.lax.sort_key_val(dst, src)
    bounds = jnp.arange(0, (NCHUNK + 1) * C, C, dtype=jnp.int32)
    eb = jnp.searchsorted(dst_s, bounds).astype(jnp.int32)
    lo = eb[:-1][:, None]
    cnt = (eb[1:] - eb[:-1])[:, None]
    s = jnp.arange(NTILE, dtype=jnp.int32)[None, :]
    s0 = lo + cnt * s // NTILE
    e0 = lo + cnt * (s + 1) // NTILE
    a = s0 & ~jnp.int32(7)
    nb = (e0 - a + (B - 1)) // B
    meta = jnp.stack([a, s0, e0, nb], axis=-1).reshape(NCHUNK * NTILE, 4)
    meta = jnp.pad(meta, ((0, 0), (0, 12)))
    srcp = jnp.pad(src_s, (0, EP - E))
    dstp = jnp.pad(dst_s, (0, EP - E))
    deg1 = jnp.searchsorted(dst_s, jnp.arange(1, N + 1, dtype=jnp.int32))
    deg0 = jnp.concatenate([jnp.zeros((1,), deg1.dtype), deg1[:-1]])
    deg = 1.0 + (deg1 - deg0).astype(jnp.float32)
    return srcp, dstp, meta, deg


def kernel(x, edge_index, Wp, bp, cw0, cb0, cw1, cb1, cw2, cb2, cw3, cb3,
           pw0, pb0, pw1, pb1, pw2, pb2, rw0, rb0, rw1, rb1):
    src = edge_index[0]
    dst = edge_index[1]
    srcp, dstp, meta, deg = _partition_edges(src, dst)

    dinv = jax.lax.rsqrt(deg)
    dinv = jnp.pad(dinv, (0, NP - N))[:, None]

    xp = jnp.pad(x, ((0, NP - N), (0, 128 - D_IN)))
    wpp = jnp.pad(Wp, ((0, 128 - D_IN), (0, 0)))

    h, g = _tc_call(
        _in_body,
        [_row_spec(128), _full_spec((128, H)), _full_spec((H,)),
         _full_spec((H, H)), _row_spec(1)],
        [H, H],
    )(xp, wpp, bp, cw0, dinv)

    for cw_next, cb in ((cw1, cb0), (cw2, cb1), (cw3, cb2)):
        acc = _sc_segsum(g, srcp, dstp, meta)
        h, g = _tc_call(
            _mid_body,
            [_row_spec(H), _row_spec(H), _row_spec(1), _full_spec((H,)),
             _full_spec((H, H))],
            [H, H],
        )(acc, h, dinv, cb, cw_next)
    acc = _sc_segsum(g, srcp, dstp, meta)

    (coords,) = _tc_call(
        _out_body,
        [_row_spec(H), _row_spec(H), _row_spec(1), _full_spec((H,)),
         _full_spec((H, H)), _full_spec((H,)),
         _full_spec((H, H // 2)), _full_spec((H // 2,)),
         _full_spec((H // 2, 2)), _full_spec((2,)),
         _full_spec((H, H // 2)), _full_spec((H // 2,)),
         _full_spec((H // 2, 1)), _full_spec((1,))],
        [2],
    )(acc, h, dinv, cb3, pw0, pb0, pw1, pb1, pw2, pb2, rw0, rb0, rw1, rb1)
    return coords[:N]


# speedup vs baseline: 12.0509x; 1.5129x over previous
import functools

import jax
import jax.numpy as jnp
from jax.experimental import pallas as pl
from jax.experimental.pallas import tpu as pltpu
from jax.experimental.pallas import tpu_sc as plsc

N = 50000
E = 800000
D_IN = 41
H = 192

ROWS = 512
NP = 51200

NSC = 2
NTILE = 16
NPASS = 4
NCHUNK = NSC * NPASS
C = 6400
RPT = C // NTILE
B = 128
EP = E + 2 * B

_INTERPRET = False


def _row_spec(width):
    return pl.BlockSpec((ROWS, width), lambda i: (i, 0))


def _full_spec(shape):
    nd = len(shape)
    return pl.BlockSpec(shape, lambda i: (0,) * nd)


def _tc_call(body, in_specs, out_width_list):
    return pl.pallas_call(
        body,
        grid=(NP // ROWS,),
        in_specs=in_specs,
        out_specs=[_row_spec(w) for w in out_width_list],
        out_shape=[jax.ShapeDtypeStruct((NP, w), jnp.float32) for w in out_width_list],
        interpret=_INTERPRET,
    )


def _in_body(x_ref, wp_ref, bp_ref, cw_ref, dinv_ref, h_ref, g_ref):
    h = jnp.dot(x_ref[...], wp_ref[...], preferred_element_type=jnp.float32) + bp_ref[...]
    h_ref[...] = h
    g_ref[...] = jnp.dot(h, cw_ref[...], preferred_element_type=jnp.float32) * dinv_ref[...]


def _mid_body(acc_ref, hprev_ref, dinv_ref, cb_ref, cw_ref, h_ref, g_ref):
    conv = acc_ref[...] * dinv_ref[...] + cb_ref[...]
    h = hprev_ref[...] + jnp.maximum(conv, 0.0)
    h_ref[...] = h
    g_ref[...] = jnp.dot(h, cw_ref[...], preferred_element_type=jnp.float32) * dinv_ref[...]


def _out_body(acc_ref, hprev_ref, dinv_ref, cb_ref,
              pw0_ref, pb0_ref, pw1_ref, pb1_ref, pw2_ref, pb2_ref,
              rw0_ref, rb0_ref, rw1_ref, rb1_ref, out_ref):
    conv = acc_ref[...] * dinv_ref[...] + cb_ref[...]
    h = hprev_ref[...] + jnp.maximum(conv, 0.0)
    p = jnp.maximum(jnp.dot(h, pw0_ref[...], preferred_element_type=jnp.float32) + pb0_ref[...], 0.0)
    p = jnp.maximum(jnp.dot(p, pw1_ref[...], preferred_element_type=jnp.float32) + pb1_ref[...], 0.0)
    p = jnp.dot(p, pw2_ref[...], preferred_element_type=jnp.float32) + pb2_ref[...]
    r = jnp.maximum(jnp.dot(h, rw0_ref[...], preferred_element_type=jnp.float32) + rb0_ref[...], 0.0)
    r = jax.nn.sigmoid(jnp.dot(r, rw1_ref[...], preferred_element_type=jnp.float32) + rb1_ref[...])
    nrm = jnp.sqrt(jnp.sum(p * p, axis=-1, keepdims=True))
    out_ref[...] = p / (nrm + 1e-8) * r


def _sc_body(g_hbm, srcp_hbm, dstp_hbm, meta_hbm, out_hbm,
             acc, meta_v, sidx0, sidx1, didx0, didx1, lidx, rows0, rows1,
             sem0, sem1):
    cid = jax.lax.axis_index("c")
    sid = jax.lax.axis_index("s")
    r0 = sid * RPT
    bufs = ((sidx0, didx0, rows0, sem0), (sidx1, didx1, rows1, sem1))
    for p in range(NPASS):
        chunk = p * NSC + cid
        base = chunk * C
        pltpu.sync_copy(g_hbm.at[pl.ds(base + r0, RPT)], acc.at[pl.ds(r0, RPT)])
        pltpu.sync_copy(meta_hbm.at[chunk * NTILE + sid], meta_v)
        plsc.subcore_barrier()
        mv = meta_v[...]
        astart = mv[0]
        estart = mv[1]
        eend = mv[2]
        nb = mv[3]

        def _stage(b, k):
            si, di, rw, sm = bufs[k]
            off = pl.multiple_of(astart + b * B, 8)
            pltpu.sync_copy(srcp_hbm.at[pl.ds(off, B)], si)
            pltpu.sync_copy(dstp_hbm.at[pl.ds(off, B)], di)
            pltpu.async_copy(g_hbm.at[si], rw, sm)

        def _drain(b, k):
            si, di, rw, sm = bufs[k]
            pltpu.make_async_copy(g_hbm.at[si], rw, sm).wait()
            off = astart + b * B
            for j in range(B // 16):
                dv = di[pl.ds(j * 16, 16)]
                pos = off + j * 16 + jax.lax.iota(jnp.int32, 16)
                valid = (pos >= estart) & (pos < eend)
                lidx[pl.ds(j * 16, 16)] = jnp.where(valid, dv - base, C)
            pltpu.sync_copy(rw, acc.at[lidx], add=True)

        @pl.when(nb > 0)
        def _():
            _stage(0, 0)

        def body(i, carry):
            b0 = 2 * i
            b1 = b0 + 1

            @pl.when(b1 < nb)
            def _():
                _stage(b1, 1)

            _drain(b0, 0)

            @pl.when(b1 < nb)
            def _():
                @pl.when(b1 + 1 < nb)
                def _():
                    _stage(b1 + 1, 0)

                _drain(b1, 1)

            return carry

        jax.lax.fori_loop(0, (nb + 1) // 2, body, 0)
        plsc.subcore_barrier()
        pltpu.sync_copy(acc.at[pl.ds(r0, RPT)], out_hbm.at[pl.ds(base + r0, RPT)])


E2 = E // NSC
ET = E2 // NTILE
NB_D = (ET + B - 1) // B
DEG_TPB = NP // NTILE
DUMP_NODE = NP - 8


def _deg_body(dst_hbm, deg_hbm, dacc, didx, ones, zbuf, sem):
    cid = jax.lax.axis_index("c")
    sid = jax.lax.axis_index("s")
    r0 = sid * DEG_TPB
    for j in range(B // 16):
        ones[pl.ds(j * 16, 16)] = jnp.full((16,), 1.0, jnp.float32)
    for j in range(DEG_TPB // 16):
        zbuf[pl.ds(j * 16, 16)] = jnp.zeros((16,), jnp.float32)
    pltpu.sync_copy(zbuf, dacc.at[pl.ds(r0, DEG_TPB)])
    plsc.subcore_barrier()
    base_e = cid * E2 + sid * ET

    def body(b, carry):
        off = pl.multiple_of(base_e + b * B, 8)
        pltpu.sync_copy(dst_hbm.at[pl.ds(off, B)], didx)
        for j in range(B // 16):
            dv = didx[pl.ds(j * 16, 16)]
            pos = off + j * 16 + jax.lax.iota(jnp.int32, 16)
            didx[pl.ds(j * 16, 16)] = jnp.where(pos < base_e + ET, dv, DUMP_NODE)
        pltpu.sync_copy(ones, dacc.at[didx], add=True)
        return carry

    jax.lax.fori_loop(0, NB_D, body, 0)
    plsc.subcore_barrier()
    pltpu.sync_copy(dacc.at[pl.ds(r0, DEG_TPB)], deg_hbm.at[cid, pl.ds(r0, DEG_TPB)])


@functools.cache
def _sc_deg_call():
  return pl.kernel(
    _deg_body,
    out_type=jax.ShapeDtypeStruct((NSC, NP), jnp.float32),
    mesh=plsc.VectorSubcoreMesh(core_axis_name="c", subcore_axis_name="s", num_cores=NSC, num_subcores=NTILE),
    scratch_types=[
        pltpu.VMEM_SHARED((NP,), jnp.float32),
        pltpu.VMEM((B,), jnp.int32),
        pltpu.VMEM((B,), jnp.float32),
        pltpu.VMEM((DEG_TPB,), jnp.float32),
        pltpu.SemaphoreType.DMA,
    ],
    compiler_params=pltpu.CompilerParams(use_tc_tiling_on_sc=False),
  )


@functools.cache
def _sc_segsum_call():
  return pl.kernel(
    _sc_body,
    out_type=jax.ShapeDtypeStruct((NP, H), jnp.float32),
    mesh=plsc.VectorSubcoreMesh(core_axis_name="c", subcore_axis_name="s", num_cores=NSC, num_subcores=NTILE),
    scratch_types=[
        pltpu.VMEM_SHARED((C + 8, H), jnp.float32),
        pltpu.VMEM((16,), jnp.int32),
        pltpu.VMEM((B,), jnp.int32),
        pltpu.VMEM((B,), jnp.int32),
        pltpu.VMEM((B,), jnp.int32),
        pltpu.VMEM((B,), jnp.int32),
        pltpu.VMEM((B,), jnp.int32),
        pltpu.VMEM((B, H), jnp.float32),
        pltpu.VMEM((B, H), jnp.float32),
        pltpu.SemaphoreType.DMA,
        pltpu.SemaphoreType.DMA,
    ],
    compiler_params=pltpu.CompilerParams(use_tc_tiling_on_sc=False),
  )


def _partition_edges(src, dst):
    packed = (dst.astype(jnp.uint32) << 16) | src.astype(jnp.uint32)
    ps = jnp.sort(packed)
    srcp = jnp.pad((ps & 0xFFFF).astype(jnp.int32), (0, EP - E))
    dstp = jnp.pad((ps >> 16).astype(jnp.int32), (0, EP - E))
    return srcp, dstp


def _edge_meta(indeg):
    csum = jnp.cumsum(indeg[:N])
    idx = jnp.minimum(jnp.arange(1, NCHUNK + 1, dtype=jnp.int32) * C, N) - 1
    eb = jnp.concatenate(
        [jnp.zeros((1,), jnp.int32), csum[idx].astype(jnp.int32)])
    lo = eb[:-1][:, None]
    cnt = (eb[1:] - eb[:-1])[:, None]
    s = jnp.arange(NTILE, dtype=jnp.int32)[None, :]
    s0 = lo + cnt * s // NTILE
    e0 = lo + cnt * (s + 1) // NTILE
    a = s0 & ~jnp.int32(7)
    nb = (e0 - a + (B - 1)) // B
    meta = jnp.stack([a, s0, e0, nb], axis=-1).reshape(NCHUNK * NTILE, 4)
    return jnp.pad(meta, ((0, 0), (0, 12)))


def kernel(x, edge_index, Wp, bp, cw0, cb0, cw1, cb1, cw2, cb2, cw3, cb3,
           pw0, pb0, pw1, pb1, pw2, pb2, rw0, rb0, rw1, rb1):
    src = edge_index[0]
    dst = edge_index[1]
    srcp, dstp = _partition_edges(src, dst)
    degp = _sc_deg_call()(jnp.pad(dst, (0, B)))
    indeg = degp[0] + degp[1]
    meta = _edge_meta(indeg)
    dinv = jax.lax.rsqrt(1.0 + indeg)[:, None]

    xp = jnp.pad(x, ((0, NP - N), (0, 128 - D_IN)))
    wpp = jnp.pad(Wp, ((0, 128 - D_IN), (0, 0)))

    h, g = _tc_call(
        _in_body,
        [_row_spec(128), _full_spec((128, H)), _full_spec((H,)),
         _full_spec((H, H)), _row_spec(1)],
        [H, H],
    )(xp, wpp, bp, cw0, dinv)

    for cw_next, cb in ((cw1, cb0), (cw2, cb1), (cw3, cb2)):
        acc = _sc_segsum_call()(g, srcp, dstp, meta)
        h, g = _tc_call(
            _mid_body,
            [_row_spec(H), _row_spec(H), _row_spec(1), _full_spec((H,)),
             _full_spec((H, H))],
            [H, H],
        )(acc, h, dinv, cb, cw_next)
    acc = _sc_segsum_call()(g, srcp, dstp, meta)

    (coords,) = _tc_call(
        _out_body,
        [_row_spec(H), _row_spec(H), _row_spec(1), _full_spec((H,)),
         _full_spec((H, H)), _full_spec((H,)),
         _full_spec((H, H // 2)), _full_spec((H // 2,)),
         _full_spec((H // 2, 2)), _full_spec((2,)),
         _full_spec((H, H // 2)), _full_spec((H // 2,)),
         _full_spec((H // 2, 1)), _full_spec((1,))],
        [2],
    )(acc, h, dinv, cb3, pw0, pb0, pw1, pb1, pw2, pb2, rw0, rb0, rw1, rb1)
    return coords[:N]
